# Initial kernel scaffold; baseline (speedup 1.0000x reference)
import jax
import jax.numpy as jnp
from jax.experimental import pallas as pl
from jax.experimental.pallas import tpu as pltpu


def _linear_body(x1_ref, x2_ref, wt_ref, b_ref, y_ref):
    h = wt_ref.shape[1]
    y1 = jnp.dot(x1_ref[:], wt_ref[:], preferred_element_type=jnp.float32)
    y2 = jnp.dot(x2_ref[:], wt_ref[:], preferred_element_type=jnp.float32)
    y_ref[:, :h] = (y1 + b_ref[:]).astype(jnp.bfloat16)
    y_ref[:, h:] = (y2 + b_ref[:]).astype(jnp.bfloat16)


def _agg_body(adj_ref, y_ref, prelu_ref, h_ref, csum_ref):
    hdim = csum_ref.shape[2]
    acc = jnp.dot(adj_ref[:].astype(jnp.bfloat16), y_ref[:],
                  preferred_element_type=jnp.float32)
    p = prelu_ref[0, 0]
    h = jnp.where(acc >= 0, acc, p * acc)
    h_ref[:] = h
    csum_ref[0, 0, :] = jnp.sum(h[:, :hdim], axis=0)


def _score_body(h_ref, csum_ref, wb_ref, bias_ref, n_ref, s1_ref, s2_ref):
    hdim = wb_ref.shape[0]
    total = jnp.sum(csum_ref[:, 0, :], axis=0, keepdims=True)
    s = jax.nn.sigmoid(total * n_ref[0, 0])
    u = jnp.dot(s, wb_ref[:].T, preferred_element_type=jnp.float32)
    bias = bias_ref[0, 0]
    s1_ref[:] = jnp.dot(h_ref[:, :hdim], u.T,
                        preferred_element_type=jnp.float32) + bias
    s2_ref[:] = jnp.dot(h_ref[:, hdim:], u.T,
                        preferred_element_type=jnp.float32) + bias


def kernel(x_1, x_2, adj, W, b, prelu_w, Wb, bias_b):
    n = adj.shape[0]
    f_in = x_1.shape[2]
    h_dim = W.shape[0]

    x1 = x_1[0]
    x2 = x_2[0]
    wt = W.T
    b2 = b.reshape(1, h_dim)
    prelu2 = prelu_w.reshape(1, 1)
    wb2 = Wb[0]
    bias2 = bias_b.reshape(1, 1)
    inv_n = jnp.full((1, 1), 1.0 / n, dtype=jnp.float32)

    bm1 = 2000
    bm2 = 400
    g1 = n // bm1
    g2 = n // bm2

    y = pl.pallas_call(
        _linear_body,
        grid=(g1,),
        in_specs=[
            pl.BlockSpec((bm1, f_in), lambda i: (i, 0)),
            pl.BlockSpec((bm1, f_in), lambda i: (i, 0)),
            pl.BlockSpec((f_in, h_dim), lambda i: (0, 0)),
            pl.BlockSpec((1, h_dim), lambda i: (0, 0)),
        ],
        out_specs=pl.BlockSpec((bm1, 2 * h_dim), lambda i: (i, 0)),
        out_shape=jax.ShapeDtypeStruct((n, 2 * h_dim), jnp.bfloat16),
        compiler_params=pltpu.CompilerParams(
            dimension_semantics=(pltpu.PARALLEL,)),
    )(x1, x2, wt, b2)

    h, csum = pl.pallas_call(
        _agg_body,
        grid=(g2,),
        in_specs=[
            pl.BlockSpec((bm2, n), lambda i: (i, 0)),
            pl.BlockSpec((n, 2 * h_dim), lambda i: (0, 0)),
            pl.BlockSpec((1, 1), lambda i: (0, 0)),
        ],
        out_specs=[
            pl.BlockSpec((bm2, 2 * h_dim), lambda i: (i, 0)),
            pl.BlockSpec((1, 1, h_dim), lambda i: (i, 0, 0)),
        ],
        out_shape=[
            jax.ShapeDtypeStruct((n, 2 * h_dim), jnp.float32),
            jax.ShapeDtypeStruct((g2, 1, h_dim), jnp.float32),
        ],
        compiler_params=pltpu.CompilerParams(
            dimension_semantics=(pltpu.PARALLEL,)),
    )(adj, y, prelu2)

    s1, s2 = pl.pallas_call(
        _score_body,
        grid=(g1,),
        in_specs=[
            pl.BlockSpec((bm1, 2 * h_dim), lambda i: (i, 0)),
            pl.BlockSpec((g2, 1, h_dim), lambda i: (0, 0, 0)),
            pl.BlockSpec((h_dim, h_dim), lambda i: (0, 0)),
            pl.BlockSpec((1, 1), lambda i: (0, 0)),
            pl.BlockSpec((1, 1), lambda i: (0, 0)),
        ],
        out_specs=[
            pl.BlockSpec((bm1, 1), lambda i: (i, 0)),
            pl.BlockSpec((bm1, 1), lambda i: (i, 0)),
        ],
        out_shape=[
            jax.ShapeDtypeStruct((n, 1), jnp.float32),
            jax.ShapeDtypeStruct((n, 1), jnp.float32),
        ],
        compiler_params=pltpu.CompilerParams(
            dimension_semantics=(pltpu.PARALLEL,)),
    )(h, csum, wb2, bias2, inv_n)

    return jnp.concatenate([s1.reshape(1, n), s2.reshape(1, n)], axis=1)


# speedup vs baseline: 1.6770x; 1.6770x over previous
import jax
import jax.numpy as jnp
from jax.experimental import pallas as pl
from jax.experimental.pallas import tpu as pltpu


def _linear_body(x1_ref, x2_ref, wt_ref, b_ref, y_ref):
    h = wt_ref.shape[1]
    y1 = jnp.dot(x1_ref[:], wt_ref[:], preferred_element_type=jnp.float32)
    y2 = jnp.dot(x2_ref[:], wt_ref[:], preferred_element_type=jnp.float32)
    y_ref[:, :h] = (y1 + b_ref[:]).astype(jnp.bfloat16)
    y_ref[:, h:] = (y2 + b_ref[:]).astype(jnp.bfloat16)


def _agg_body(adj_ref, y_ref, prelu_ref, h_ref, csum_ref):
    hdim = csum_ref.shape[2]
    acc = jnp.dot(adj_ref[:].astype(jnp.bfloat16), y_ref[:],
                  preferred_element_type=jnp.float32)
    p = prelu_ref[0, 0]
    h = jnp.where(acc >= 0, acc, p * acc)
    h_ref[:] = h
    csum_ref[0, 0, :] = jnp.sum(h[:, :hdim], axis=0)


def _score_body(h_ref, csum_ref, wb_ref, bias_ref, n_ref, s1_ref, s2_ref):
    hdim = wb_ref.shape[0]
    total = jnp.sum(csum_ref[:, 0, :], axis=0, keepdims=True)
    s = jax.nn.sigmoid(total * n_ref[0, 0])
    u = jnp.dot(s, wb_ref[:].T, preferred_element_type=jnp.float32)
    bias = bias_ref[0, 0]
    s1_ref[:] = jnp.sum(h_ref[:, :hdim] * u, axis=1, keepdims=True) + bias
    s2_ref[:] = jnp.sum(h_ref[:, hdim:] * u, axis=1, keepdims=True) + bias


def kernel(x_1, x_2, adj, W, b, prelu_w, Wb, bias_b):
    n = adj.shape[0]
    f_in = x_1.shape[2]
    h_dim = W.shape[0]

    x1 = x_1[0]
    x2 = x_2[0]
    wt = W.T
    b2 = b.reshape(1, h_dim)
    prelu2 = prelu_w.reshape(1, 1)
    wb2 = Wb[0]
    bias2 = bias_b.reshape(1, 1)
    inv_n = jnp.full((1, 1), 1.0 / n, dtype=jnp.float32)

    bm1 = 2000
    bm2 = 400
    g1 = n // bm1
    g2 = n // bm2

    y = pl.pallas_call(
        _linear_body,
        grid=(g1,),
        in_specs=[
            pl.BlockSpec((bm1, f_in), lambda i: (i, 0)),
            pl.BlockSpec((bm1, f_in), lambda i: (i, 0)),
            pl.BlockSpec((f_in, h_dim), lambda i: (0, 0)),
            pl.BlockSpec((1, h_dim), lambda i: (0, 0)),
        ],
        out_specs=pl.BlockSpec((bm1, 2 * h_dim), lambda i: (i, 0)),
        out_shape=jax.ShapeDtypeStruct((n, 2 * h_dim), jnp.bfloat16),
        compiler_params=pltpu.CompilerParams(
            dimension_semantics=(pltpu.PARALLEL,)),
    )(x1, x2, wt, b2)

    h, csum = pl.pallas_call(
        _agg_body,
        grid=(g2,),
        in_specs=[
            pl.BlockSpec((bm2, n), lambda i: (i, 0)),
            pl.BlockSpec((n, 2 * h_dim), lambda i: (0, 0)),
            pl.BlockSpec((1, 1), lambda i: (0, 0)),
        ],
        out_specs=[
            pl.BlockSpec((bm2, 2 * h_dim), lambda i: (i, 0)),
            pl.BlockSpec((1, 1, h_dim), lambda i: (i, 0, 0)),
        ],
        out_shape=[
            jax.ShapeDtypeStruct((n, 2 * h_dim), jnp.float32),
            jax.ShapeDtypeStruct((g2, 1, h_dim), jnp.float32),
        ],
        compiler_params=pltpu.CompilerParams(
            dimension_semantics=(pltpu.PARALLEL,)),
    )(adj, y, prelu2)

    s1, s2 = pl.pallas_call(
        _score_body,
        grid=(g1,),
        in_specs=[
            pl.BlockSpec((bm1, 2 * h_dim), lambda i: (i, 0)),
            pl.BlockSpec((g2, 1, h_dim), lambda i: (0, 0, 0)),
            pl.BlockSpec((h_dim, h_dim), lambda i: (0, 0)),
            pl.BlockSpec((1, 1), lambda i: (0, 0)),
            pl.BlockSpec((1, 1), lambda i: (0, 0)),
        ],
        out_specs=[
            pl.BlockSpec((bm1, 1), lambda i: (i, 0)),
            pl.BlockSpec((bm1, 1), lambda i: (i, 0)),
        ],
        out_shape=[
            jax.ShapeDtypeStruct((n, 1), jnp.float32),
            jax.ShapeDtypeStruct((n, 1), jnp.float32),
        ],
        compiler_params=pltpu.CompilerParams(
            dimension_semantics=(pltpu.PARALLEL,)),
    )(h, csum, wb2, bias2, inv_n)

    return jnp.concatenate([s1.reshape(1, n), s2.reshape(1, n)], axis=1)


# speedup vs baseline: 1.6842x; 1.0043x over previous
import jax
import jax.numpy as jnp
from jax.experimental import pallas as pl
from jax.experimental.pallas import tpu as pltpu


def _linear_body(x1_ref, x2_ref, wt_ref, b_ref, y_ref):
    h = wt_ref.shape[1]
    y1 = jnp.dot(x1_ref[:], wt_ref[:], preferred_element_type=jnp.float32)
    y2 = jnp.dot(x2_ref[:], wt_ref[:], preferred_element_type=jnp.float32)
    y_ref[:, :h] = (y1 + b_ref[:]).astype(jnp.bfloat16)
    y_ref[:, h:] = (y2 + b_ref[:]).astype(jnp.bfloat16)


def _agg_body(adj_a_ref, adj_b_ref, y_ref, prelu_ref, h_ref, csum_ref):
    hdim = csum_ref.shape[2]
    m = adj_a_ref.shape[0]
    p = prelu_ref[0, 0]
    acc_a = jnp.dot(adj_a_ref[:].astype(jnp.bfloat16), y_ref[:],
                    preferred_element_type=jnp.float32)
    h_a = jnp.where(acc_a >= 0, acc_a, p * acc_a)
    h_ref[:m, :] = h_a
    acc_b = jnp.dot(adj_b_ref[:].astype(jnp.bfloat16), y_ref[:],
                    preferred_element_type=jnp.float32)
    h_b = jnp.where(acc_b >= 0, acc_b, p * acc_b)
    h_ref[m:, :] = h_b
    csum_ref[0, 0, :] = (jnp.sum(h_a[:, :hdim], axis=0)
                         + jnp.sum(h_b[:, :hdim], axis=0))


def _score_body(h_ref, csum_ref, wb_ref, bias_ref, n_ref, s1_ref, s2_ref):
    hdim = wb_ref.shape[0]
    total = jnp.sum(csum_ref[:, 0, :], axis=0, keepdims=True)
    s = jax.nn.sigmoid(total * n_ref[0, 0])
    u = jnp.dot(s, wb_ref[:].T, preferred_element_type=jnp.float32)
    bias = bias_ref[0, 0]
    s1_ref[:] = jnp.sum(h_ref[:, :hdim] * u, axis=1, keepdims=True) + bias
    s2_ref[:] = jnp.sum(h_ref[:, hdim:] * u, axis=1, keepdims=True) + bias


def kernel(x_1, x_2, adj, W, b, prelu_w, Wb, bias_b):
    n = adj.shape[0]
    f_in = x_1.shape[2]
    h_dim = W.shape[0]

    x1 = x_1[0]
    x2 = x_2[0]
    wt = W.T
    b2 = b.reshape(1, h_dim)
    prelu2 = prelu_w.reshape(1, 1)
    wb2 = Wb[0]
    bias2 = bias_b.reshape(1, 1)
    inv_n = jnp.full((1, 1), 1.0 / n, dtype=jnp.float32)

    bm1 = 2000
    bm2 = 400
    g1 = n // bm1
    g2 = n // bm2

    y = pl.pallas_call(
        _linear_body,
        grid=(g1,),
        in_specs=[
            pl.BlockSpec((bm1, f_in), lambda i: (i, 0)),
            pl.BlockSpec((bm1, f_in), lambda i: (i, 0)),
            pl.BlockSpec((f_in, h_dim), lambda i: (0, 0)),
            pl.BlockSpec((1, h_dim), lambda i: (0, 0)),
        ],
        out_specs=pl.BlockSpec((bm1, 2 * h_dim), lambda i: (i, 0)),
        out_shape=jax.ShapeDtypeStruct((n, 2 * h_dim), jnp.bfloat16),
        compiler_params=pltpu.CompilerParams(
            dimension_semantics=(pltpu.PARALLEL,)),
    )(x1, x2, wt, b2)

    h, csum = pl.pallas_call(
        _agg_body,
        grid=(g2,),
        in_specs=[
            pl.BlockSpec((bm2 // 2, n), lambda i: (2 * i, 0)),
            pl.BlockSpec((bm2 // 2, n), lambda i: (2 * i + 1, 0)),
            pl.BlockSpec((n, 2 * h_dim), lambda i: (0, 0)),
            pl.BlockSpec((1, 1), lambda i: (0, 0)),
        ],
        out_specs=[
            pl.BlockSpec((bm2, 2 * h_dim), lambda i: (i, 0)),
            pl.BlockSpec((1, 1, h_dim), lambda i: (i, 0, 0)),
        ],
        out_shape=[
            jax.ShapeDtypeStruct((n, 2 * h_dim), jnp.float32),
            jax.ShapeDtypeStruct((g2, 1, h_dim), jnp.float32),
        ],
        compiler_params=pltpu.CompilerParams(
            dimension_semantics=(pltpu.PARALLEL,)),
    )(adj, adj, y, prelu2)

    s1, s2 = pl.pallas_call(
        _score_body,
        grid=(g1,),
        in_specs=[
            pl.BlockSpec((bm1, 2 * h_dim), lambda i: (i, 0)),
            pl.BlockSpec((g2, 1, h_dim), lambda i: (0, 0, 0)),
            pl.BlockSpec((h_dim, h_dim), lambda i: (0, 0)),
            pl.BlockSpec((1, 1), lambda i: (0, 0)),
            pl.BlockSpec((1, 1), lambda i: (0, 0)),
        ],
        out_specs=[
            pl.BlockSpec((bm1, 1), lambda i: (i, 0)),
            pl.BlockSpec((bm1, 1), lambda i: (i, 0)),
        ],
        out_shape=[
            jax.ShapeDtypeStruct((n, 1), jnp.float32),
            jax.ShapeDtypeStruct((n, 1), jnp.float32),
        ],
        compiler_params=pltpu.CompilerParams(
            dimension_semantics=(pltpu.PARALLEL,)),
    )(h, csum, wb2, bias2, inv_n)

    return jnp.concatenate([s1.reshape(1, n), s2.reshape(1, n)], axis=1)


# speedup vs baseline: 1.6974x; 1.0078x over previous
import jax
import jax.numpy as jnp
from jax.experimental import pallas as pl
from jax.experimental.pallas import tpu as pltpu


def _linear_body(x1_ref, x2_ref, wt_ref, b_ref, y_ref):
    h = wt_ref.shape[1]
    y1 = jnp.dot(x1_ref[:], wt_ref[:], preferred_element_type=jnp.float32)
    y2 = jnp.dot(x2_ref[:], wt_ref[:], preferred_element_type=jnp.float32)
    y_ref[:, :h] = (y1 + b_ref[:]).astype(jnp.bfloat16)
    y_ref[:, h:] = (y2 + b_ref[:]).astype(jnp.bfloat16)


def _agg_body(adj_a_ref, adj_b_ref, y_ref, prelu_ref, h_ref, csum_ref):
    hdim = csum_ref.shape[2]
    m = adj_a_ref.shape[0]
    p = prelu_ref[0, 0]
    acc_a = jnp.dot(adj_a_ref[:].astype(jnp.bfloat16), y_ref[:],
                    preferred_element_type=jnp.float32)
    h_a = jnp.where(acc_a >= 0, acc_a, p * acc_a)
    h_ref[:m, :] = h_a.astype(jnp.bfloat16)
    acc_b = jnp.dot(adj_b_ref[:].astype(jnp.bfloat16), y_ref[:],
                    preferred_element_type=jnp.float32)
    h_b = jnp.where(acc_b >= 0, acc_b, p * acc_b)
    h_ref[m:, :] = h_b.astype(jnp.bfloat16)
    csum_ref[0, 0, :] = (jnp.sum(h_a[:, :hdim], axis=0)
                         + jnp.sum(h_b[:, :hdim], axis=0))


def _score_body(h_ref, csum_ref, wb_ref, bias_ref, n_ref, s1_ref, s2_ref):
    hdim = wb_ref.shape[0]
    total = jnp.sum(csum_ref[:, 0, :], axis=0, keepdims=True)
    s = jax.nn.sigmoid(total * n_ref[0, 0])
    u = jnp.dot(s, wb_ref[:].T, preferred_element_type=jnp.float32)
    bias = bias_ref[0, 0]
    s1_ref[:] = jnp.sum(h_ref[:, :hdim] * u, axis=1, keepdims=True) + bias
    s2_ref[:] = jnp.sum(h_ref[:, hdim:] * u, axis=1, keepdims=True) + bias


def kernel(x_1, x_2, adj, W, b, prelu_w, Wb, bias_b):
    n = adj.shape[0]
    f_in = x_1.shape[2]
    h_dim = W.shape[0]

    x1 = x_1[0]
    x2 = x_2[0]
    wt = W.T
    b2 = b.reshape(1, h_dim)
    prelu2 = prelu_w.reshape(1, 1)
    wb2 = Wb[0]
    bias2 = bias_b.reshape(1, 1)
    inv_n = jnp.full((1, 1), 1.0 / n, dtype=jnp.float32)

    bm1 = 2000
    bm2 = 400
    g1 = n // bm1
    g2 = n // bm2

    y = pl.pallas_call(
        _linear_body,
        grid=(g1,),
        in_specs=[
            pl.BlockSpec((bm1, f_in), lambda i: (i, 0)),
            pl.BlockSpec((bm1, f_in), lambda i: (i, 0)),
            pl.BlockSpec((f_in, h_dim), lambda i: (0, 0)),
            pl.BlockSpec((1, h_dim), lambda i: (0, 0)),
        ],
        out_specs=pl.BlockSpec((bm1, 2 * h_dim), lambda i: (i, 0)),
        out_shape=jax.ShapeDtypeStruct((n, 2 * h_dim), jnp.bfloat16),
        compiler_params=pltpu.CompilerParams(
            dimension_semantics=(pltpu.PARALLEL,)),
    )(x1, x2, wt, b2)

    h, csum = pl.pallas_call(
        _agg_body,
        grid=(g2,),
        in_specs=[
            pl.BlockSpec((bm2 // 2, n), lambda i: (2 * i, 0)),
            pl.BlockSpec((bm2 // 2, n), lambda i: (2 * i + 1, 0)),
            pl.BlockSpec((n, 2 * h_dim), lambda i: (0, 0)),
            pl.BlockSpec((1, 1), lambda i: (0, 0)),
        ],
        out_specs=[
            pl.BlockSpec((bm2, 2 * h_dim), lambda i: (i, 0)),
            pl.BlockSpec((1, 1, h_dim), lambda i: (i, 0, 0)),
        ],
        out_shape=[
            jax.ShapeDtypeStruct((n, 2 * h_dim), jnp.bfloat16),
            jax.ShapeDtypeStruct((g2, 1, h_dim), jnp.float32),
        ],
        compiler_params=pltpu.CompilerParams(
            dimension_semantics=(pltpu.PARALLEL,)),
    )(adj, adj, y, prelu2)

    s1, s2 = pl.pallas_call(
        _score_body,
        grid=(g1,),
        in_specs=[
            pl.BlockSpec((bm1, 2 * h_dim), lambda i: (i, 0)),
            pl.BlockSpec((g2, 1, h_dim), lambda i: (0, 0, 0)),
            pl.BlockSpec((h_dim, h_dim), lambda i: (0, 0)),
            pl.BlockSpec((1, 1), lambda i: (0, 0)),
            pl.BlockSpec((1, 1), lambda i: (0, 0)),
        ],
        out_specs=[
            pl.BlockSpec((bm1, 1), lambda i: (i, 0)),
            pl.BlockSpec((bm1, 1), lambda i: (i, 0)),
        ],
        out_shape=[
            jax.ShapeDtypeStruct((n, 1), jnp.float32),
            jax.ShapeDtypeStruct((n, 1), jnp.float32),
        ],
        compiler_params=pltpu.CompilerParams(
            dimension_semantics=(pltpu.PARALLEL,)),
    )(h, csum, wb2, bias2, inv_n)

    return jnp.concatenate([s1.reshape(1, n), s2.reshape(1, n)], axis=1)
